# Initial kernel scaffold; baseline (speedup 1.0000x reference)
import jax
import jax.numpy as jnp
from jax.experimental import pallas as pl


def kernel(x, ease_scores, table):
    raise NotImplementedError("write your pallas kernel here")


# speedup vs baseline: 1.4970x; 1.4970x over previous
import functools

import jax
import jax.numpy as jnp
from jax import lax
from jax.experimental import pallas as pl
from jax.experimental.pallas import tpu as pltpu
from jax.experimental.pallas import tpu_sc as plsc

VOCAB = 1000000
D_MODEL = 32
BATCH = 16384
HIST = 50

N_ROWS = BATCH * HIST
N_ELEMS = N_ROWS * D_MODEL

NUM_CORES = 2
NUM_SUBCORES = 16
NW = NUM_CORES * NUM_SUBCORES
ROWS_PER_W = N_ROWS // NW
IDX_W = 128
K_GATHERS = 20
CHUNK_ROWS = IDX_W * K_GATHERS
N_OUTER = ROWS_PER_W // CHUNK_ROWS
IDX_ROWS_PER_W = ROWS_PER_W // IDX_W

LANES = 128
TC_ROWS = N_ELEMS // LANES
TC_BLOCK = 1024
TC_GRID = TC_ROWS // TC_BLOCK


def _sc_gather(x_flat, table):
    mesh = plsc.VectorSubcoreMesh(core_axis_name="c", subcore_axis_name="s")

    @functools.partial(
        pl.kernel,
        mesh=mesh,
        out_type=jax.ShapeDtypeStruct((N_ROWS, D_MODEL), jnp.float32),
        scratch_types=[
            pltpu.VMEM((CHUNK_ROWS,), jnp.int32),
            pltpu.VMEM((CHUNK_ROWS, D_MODEL), jnp.float32),
            pltpu.SemaphoreType.DMA,
        ],
        compiler_params=pltpu.CompilerParams(use_tc_tiling_on_sc=False),
    )
    def k(x_hbm, table_hbm, out_hbm, idx_v, rows_v, sem):
        wid = lax.axis_index("s") * NUM_CORES + lax.axis_index("c")

        def step(g, _):
            row0 = wid * ROWS_PER_W + g * CHUNK_ROWS
            pltpu.sync_copy(x_hbm.at[pl.ds(row0, CHUNK_ROWS)], idx_v)
            copies = [
                pltpu.async_copy(
                    table_hbm.at[idx_v.at[pl.ds(j * IDX_W, IDX_W)]],
                    rows_v.at[pl.ds(j * IDX_W, IDX_W)],
                    sem,
                )
                for j in range(K_GATHERS)
            ]
            for c in copies:
                c.wait()
            pltpu.sync_copy(rows_v, out_hbm.at[pl.ds(row0, CHUNK_ROWS)])
            return _

        lax.fori_loop(0, N_OUTER, step, None)

    return k(x_flat, table)


def _threefry_bits(e):
    ks0 = jnp.uint32(0)
    ks1 = jnp.uint32(42)
    ks2 = jnp.uint32(0x1BD11BDA) ^ ks0 ^ ks1
    ks = (ks0, ks1, ks2)
    rot = ((13, 15, 26, 6), (17, 29, 16, 24))
    x0 = jnp.zeros_like(e) + ks[0]
    x1 = e + ks[1]
    for i in range(5):
        for r in rot[i % 2]:
            x0 = x0 + x1
            x1 = (x1 << r) | (x1 >> (32 - r))
            x1 = x1 ^ x0
        x0 = x0 + ks[(i + 1) % 3]
        x1 = x1 + ks[(i + 2) % 3] + jnp.uint32(i + 1)
    return x0 ^ x1


_U_LO = -0.9999999403953552
_SQRT2 = 1.4142135623730951


def _tc_body(ease_ref, emb_ref, out_ref, avg_ref):
    avg = jnp.sum(ease_ref[0, :]) * jnp.float32(0.125)
    s_noise = jnp.float32(0.2) * (jnp.float32(1.0) - avg)
    s_out = jnp.float32(0.5) + avg

    i = pl.program_id(0)
    r = lax.broadcasted_iota(jnp.int32, (TC_BLOCK, LANES), 0)
    c = lax.broadcasted_iota(jnp.int32, (TC_BLOCK, LANES), 1)
    e = (((i * TC_BLOCK + r) << 7) + c).astype(jnp.uint32)
    bits = _threefry_bits(e)
    fb = (bits >> 9) | jnp.uint32(0x3F800000)
    f = lax.bitcast_convert_type(fb, jnp.float32) - jnp.float32(1.0)
    u = jnp.maximum(jnp.float32(_U_LO),
                    f * jnp.float32(1.0 - _U_LO) + jnp.float32(_U_LO))
    noise = jnp.float32(_SQRT2) * lax.erf_inv(u)

    out_ref[...] = (emb_ref[...] + noise * s_noise) * s_out
    avg_ref[...] = jnp.reshape(avg, (1, 1))


def _tc_combine(ease2, emb2d):
    return pl.pallas_call(
        _tc_body,
        grid=(TC_GRID,),
        in_specs=[
            pl.BlockSpec((1, 8), lambda i: (0, 0)),
            pl.BlockSpec((TC_BLOCK, LANES), lambda i: (i, 0)),
        ],
        out_specs=[
            pl.BlockSpec((TC_BLOCK, LANES), lambda i: (i, 0)),
            pl.BlockSpec((1, 1), lambda i: (0, 0)),
        ],
        out_shape=[
            jax.ShapeDtypeStruct((TC_ROWS, LANES), jnp.float32),
            jax.ShapeDtypeStruct((1, 1), jnp.float32),
        ],
        compiler_params=pltpu.CompilerParams(
            dimension_semantics=("arbitrary",),
        ),
    )(ease2, emb2d)


def kernel(x, ease_scores, table):
    x_flat = x.astype(jnp.int32).reshape(N_ROWS)
    emb = _sc_gather(x_flat, table)
    emb2d = emb.reshape(TC_ROWS, LANES)
    out2d, avg = _tc_combine(ease_scores.reshape(1, 8), emb2d)
    return out2d.reshape(BATCH, HIST, D_MODEL), avg.reshape(())


# speedup vs baseline: 1.8449x; 1.2324x over previous
import functools

import jax
import jax.numpy as jnp
from jax import lax
from jax.experimental import pallas as pl
from jax.experimental.pallas import tpu as pltpu
from jax.experimental.pallas import tpu_sc as plsc

VOCAB = 1000000
D_MODEL = 32
BATCH = 16384
HIST = 50

N_ROWS = BATCH * HIST
N_ELEMS = N_ROWS * D_MODEL

NUM_CORES = 2
NUM_SUBCORES = 16
NW = NUM_CORES * NUM_SUBCORES
ROWS_PER_W = N_ROWS // NW
IDX_W = 128
K_GATHERS = 20
CHUNK_ROWS = IDX_W * K_GATHERS
N_OUTER = ROWS_PER_W // CHUNK_ROWS
IDX_ROWS_PER_W = ROWS_PER_W // IDX_W

LANES = 128
TC_ROWS = N_ELEMS // LANES
TC_BLOCK = 1024
TC_GRID = TC_ROWS // TC_BLOCK


def _sc_gather(x_flat, table):
    mesh = plsc.VectorSubcoreMesh(core_axis_name="c", subcore_axis_name="s")

    @functools.partial(
        pl.kernel,
        mesh=mesh,
        out_type=jax.ShapeDtypeStruct((N_ROWS, D_MODEL), jnp.float32),
        scratch_types=[
            pltpu.VMEM((CHUNK_ROWS,), jnp.int32),
            pltpu.VMEM((CHUNK_ROWS, D_MODEL), jnp.float32),
            pltpu.SemaphoreType.DMA,
        ],
        compiler_params=pltpu.CompilerParams(use_tc_tiling_on_sc=False),
    )
    def k(x_hbm, table_hbm, out_hbm, idx_v, rows_v, sem):
        wid = lax.axis_index("s") * NUM_CORES + lax.axis_index("c")

        def step(g, _):
            row0 = wid * ROWS_PER_W + g * CHUNK_ROWS
            pltpu.sync_copy(x_hbm.at[pl.ds(row0, CHUNK_ROWS)], idx_v)
            copies = [
                pltpu.async_copy(
                    table_hbm.at[idx_v.at[pl.ds(j * IDX_W, IDX_W)]],
                    rows_v.at[pl.ds(j * IDX_W, IDX_W)],
                    sem,
                )
                for j in range(K_GATHERS)
            ]
            for c in copies:
                c.wait()
            pltpu.sync_copy(rows_v, out_hbm.at[pl.ds(row0, CHUNK_ROWS)])
            return _

        lax.fori_loop(0, N_OUTER, step, None)

    return k(x_flat, table)


def _threefry_bits(e):
    ks0 = jnp.uint32(0)
    ks1 = jnp.uint32(42)
    ks2 = jnp.uint32(0x1BD11BDA) ^ ks0 ^ ks1
    ks = (ks0, ks1, ks2)
    rot = ((13, 15, 26, 6), (17, 29, 16, 24))
    x0 = jnp.zeros_like(e) + ks[0]
    x1 = e + ks[1]
    for i in range(5):
        for r in rot[i % 2]:
            x0 = x0 + x1
            x1 = (x1 << r) | (x1 >> (32 - r))
            x1 = x1 ^ x0
        x0 = x0 + ks[(i + 1) % 3]
        x1 = x1 + ks[(i + 2) % 3] + jnp.uint32(i + 1)
    return x0 ^ x1


_U_LO = -0.9999999403953552
_SQRT2 = 1.4142135623730951


SUB = 64
UNROLL = 2
STEP_ROWS = SUB * UNROLL
N_SUB = TC_BLOCK // STEP_ROWS


def _tc_body(ease_ref, emb_ref, out_ref, avg_ref):
    avg = jnp.sum(ease_ref[0, :]) * jnp.float32(0.125)
    s_noise = jnp.float32(0.2) * (jnp.float32(1.0) - avg)
    s_out = jnp.float32(0.5) + avg

    i = pl.program_id(0)
    r = lax.broadcasted_iota(jnp.int32, (SUB, LANES), 0)
    c = lax.broadcasted_iota(jnp.int32, (SUB, LANES), 1)
    lin = (r << 7) + c

    def sub(k, _):
        for u_ in range(UNROLL):
            row0 = k * STEP_ROWS + u_ * SUB
            e0 = (i * TC_BLOCK + row0) << 7
            e = (e0 + lin).astype(jnp.uint32)
            bits = _threefry_bits(e)
            fb = (bits >> 9) | jnp.uint32(0x3F800000)
            f = lax.bitcast_convert_type(fb, jnp.float32) - jnp.float32(1.0)
            u = jnp.maximum(jnp.float32(_U_LO),
                            f * jnp.float32(1.0 - _U_LO) + jnp.float32(_U_LO))
            noise = jnp.float32(_SQRT2) * lax.erf_inv(u)
            out_ref[pl.ds(row0, SUB), :] = (
                (emb_ref[pl.ds(row0, SUB), :] + noise * s_noise) * s_out)
        return _

    lax.fori_loop(0, N_SUB, sub, None)
    avg_ref[...] = jnp.reshape(avg, (1, 1))


def _tc_combine(ease2, emb2d):
    return pl.pallas_call(
        _tc_body,
        grid=(TC_GRID,),
        in_specs=[
            pl.BlockSpec((1, 8), lambda i: (0, 0)),
            pl.BlockSpec((TC_BLOCK, LANES), lambda i: (i, 0)),
        ],
        out_specs=[
            pl.BlockSpec((TC_BLOCK, LANES), lambda i: (i, 0)),
            pl.BlockSpec((1, 1), lambda i: (0, 0)),
        ],
        out_shape=[
            jax.ShapeDtypeStruct((TC_ROWS, LANES), jnp.float32),
            jax.ShapeDtypeStruct((1, 1), jnp.float32),
        ],
        compiler_params=pltpu.CompilerParams(
            dimension_semantics=("arbitrary",),
        ),
    )(ease2, emb2d)


def kernel(x, ease_scores, table):
    x_flat = x.astype(jnp.int32).reshape(N_ROWS)
    emb = _sc_gather(x_flat, table)
    emb2d = emb.reshape(TC_ROWS, LANES)
    out2d, avg = _tc_combine(ease_scores.reshape(1, 8), emb2d)
    return out2d.reshape(BATCH, HIST, D_MODEL), avg.reshape(())
